# baseline (speedup 1.0000x reference)
import functools

import jax
import jax.numpy as jnp
from jax import lax
from jax.experimental import pallas as pl
from jax.experimental.pallas import tpu as pltpu
from jax.experimental.pallas import tpu_sc as plsc

B = 16384
NUM_CARDS = 52
EMB = 16
OUT = 16
NOUT = 12
SLOT_STRIDE = 56
NSLOTS = 7
NW = 32
BPW = B // NW


def _fold_body(emb_ref, wht1_ref, wht2_ref, wwr1_ref, wwr2_ref,
               wph1_ref, wph2_ref, bht1_ref, bwr1_ref, bph1_ref, b2_ref,
               table_ref):
    f32 = jnp.float32
    i32 = jnp.int32

    r16_10 = lax.broadcasted_iota(i32, (16, 10), 0)
    c16_10 = lax.broadcasted_iota(i32, (16, 10), 1)
    s_ht = (r16_10 == c16_10).astype(f32)
    riota = lax.broadcasted_iota(i32, (16, 1024), 0)

    def matmul(a, b):
        return lax.dot_general(a, b, (((1,), (0,)), ((), ())),
                               precision=lax.Precision.HIGHEST,
                               preferred_element_type=f32)

    def matmul_t(a, b):
        return lax.dot_general(a, b, (((1,), (1,)), ((), ())),
                               precision=lax.Precision.HIGHEST,
                               preferred_element_type=f32)

    w2p_ht = matmul(s_ht, wht2_ref[...])
    w2p_wr = jnp.where(riota == 10,
                       jnp.broadcast_to(wwr2_ref[...], (16, 1024)), 0.0)
    w2p_ph = jnp.where(riota == 11,
                       jnp.broadcast_to(wph2_ref[...], (16, 1024)), 0.0)

    w_all = (matmul(w2p_ht, wht1_ref[...])
             + matmul(w2p_wr, wwr1_ref[...])
             + matmul(w2p_ph, wph1_ref[...]))

    b_all = (b2_ref[...]
             + matmul_t(bht1_ref[...], w2p_ht)
             + matmul_t(bwr1_ref[...], w2p_wr)
             + matmul_t(bph1_ref[...], w2p_ph))

    emb = emb_ref[...]
    for j in range(NSLOTS):
        t_j = matmul_t(emb, w_all[:, EMB * j:EMB * (j + 1)])
        if j == 0:
            t_j = t_j + b_all
        table_ref[pl.ds(SLOT_STRIDE * j, NUM_CARDS), :] = t_j


def _sc_body(table_hbm, hand_hbm, pub_hbm, out_hbm,
             table_v, hidx_v, pidx_v, out_v, sem):
    wid = lax.axis_index("s") * 2 + lax.axis_index("c")
    nh = 2 * BPW
    np_ = 5 * BPW

    c1 = pltpu.async_copy(table_hbm, table_v, sem)
    c2 = pltpu.async_copy(hand_hbm.at[pl.ds(wid * nh, nh)], hidx_v, sem)
    c3 = pltpu.async_copy(pub_hbm.at[pl.ds(wid * np_, np_)], pidx_v, sem)
    c1.wait()
    c2.wait()
    c3.wait()

    iota16 = lax.iota(jnp.int32, 16)

    def group(g, carry):
        rows16 = iota16 + g * 16
        slots = [plsc.load_gather(hidx_v, [rows16 * 2]),
                 plsc.load_gather(hidx_v, [rows16 * 2 + 1]) + SLOT_STRIDE]
        for k in range(5):
            slots.append(plsc.load_gather(pidx_v, [rows16 * 5 + k])
                         + SLOT_STRIDE * (2 + k))
        for c in range(NOUT):
            cc = jnp.full((16,), c, jnp.int32)
            acc = plsc.load_gather(table_v, [slots[0], cc])
            for s in slots[1:]:
                acc = acc + plsc.load_gather(table_v, [s, cc])
            plsc.store_scatter(out_v, [rows16, cc], acc)
        return carry

    lax.fori_loop(0, BPW // 16, group, 0)
    pltpu.sync_copy(out_v, out_hbm.at[pl.ds(wid * BPW, BPW)])


@functools.cache
def _sc_gather_sum():
    return pl.kernel(
        _sc_body,
        out_type=jax.ShapeDtypeStruct((B, OUT), jnp.float32),
        mesh=plsc.VectorSubcoreMesh(core_axis_name="c", subcore_axis_name="s"),
        scratch_types=[
            pltpu.VMEM((SLOT_STRIDE * 8, OUT), jnp.float32),
            pltpu.VMEM((2 * BPW,), jnp.int32),
            pltpu.VMEM((5 * BPW,), jnp.int32),
            pltpu.VMEM((BPW, OUT), jnp.float32),
            pltpu.SemaphoreType.DMA,
        ],
        compiler_params=pltpu.CompilerParams(use_tc_tiling_on_sc=False,
                                             needs_layout_passes=False,
                                             skip_device_barrier=True),
    )


def kernel(hand_indices, public_indices, embedding,
           W_ht1, b_ht1, W_ht2, b_ht2,
           W_wr1, b_wr1, W_wr2, b_wr2,
           W_ph1, b_ph1, W_ph2, b_ph2):
    i32 = jnp.int32
    hand_flat = hand_indices.astype(i32).reshape(-1)
    pub_flat = public_indices.astype(i32).reshape(-1)

    table = pl.pallas_call(
        _fold_body,
        out_shape=jax.ShapeDtypeStruct((SLOT_STRIDE * 8, OUT), jnp.float32),
    )(embedding, W_ht1, W_ht2, W_wr1, W_wr2, W_ph1, W_ph2,
      b_ht1.reshape(1, -1), b_wr1.reshape(1, -1), b_ph1.reshape(1, -1),
      jnp.concatenate([b_ht2, b_wr2, b_ph2,
                       jnp.zeros((4,), jnp.float32)]).reshape(1, OUT))

    out = _sc_gather_sum()(table, hand_flat, pub_flat)
    return (out[:, :10], out[:, 10:11], out[:, 11:12])


# speedup vs baseline: 1.0276x; 1.0276x over previous
import functools

import jax
import jax.numpy as jnp
from jax import lax
from jax.experimental import pallas as pl
from jax.experimental.pallas import tpu as pltpu
from jax.experimental.pallas import tpu_sc as plsc

B = 16384
NUM_CARDS = 52
EMB = 16
OUT = 16
NOUT = 12
SLOT_STRIDE = 56
NSLOTS = 7
NW = 32
BPW = B // NW


def _fold_body(emb_ref, wht1_ref, wht2_ref, wwr1_ref, wwr2_ref,
               wph1_ref, wph2_ref, bht1_ref, bwr1_ref, bph1_ref, b2_ref,
               table_ref):
    f32 = jnp.float32
    i32 = jnp.int32

    r16_10 = lax.broadcasted_iota(i32, (16, 10), 0)
    c16_10 = lax.broadcasted_iota(i32, (16, 10), 1)
    s_ht = (r16_10 == c16_10).astype(f32)
    riota = lax.broadcasted_iota(i32, (16, 1024), 0)

    def matmul(a, b):
        return lax.dot_general(a, b, (((1,), (0,)), ((), ())),
                               precision=lax.Precision.HIGHEST,
                               preferred_element_type=f32)

    def matmul_t(a, b):
        return lax.dot_general(a, b, (((1,), (1,)), ((), ())),
                               precision=lax.Precision.HIGHEST,
                               preferred_element_type=f32)

    w2p_ht = matmul(s_ht, wht2_ref[...])
    w2p_wr = jnp.where(riota == 10,
                       jnp.broadcast_to(wwr2_ref[...], (16, 1024)), 0.0)
    w2p_ph = jnp.where(riota == 11,
                       jnp.broadcast_to(wph2_ref[...], (16, 1024)), 0.0)

    w_all = (matmul(w2p_ht, wht1_ref[...])
             + matmul(w2p_wr, wwr1_ref[...])
             + matmul(w2p_ph, wph1_ref[...]))

    b_all = (b2_ref[...]
             + matmul_t(bht1_ref[...], w2p_ht)
             + matmul_t(bwr1_ref[...], w2p_wr)
             + matmul_t(bph1_ref[...], w2p_ph))

    emb = emb_ref[...]
    for j in range(NSLOTS):
        t_j = matmul_t(emb, w_all[:, EMB * j:EMB * (j + 1)])
        if j == 0:
            t_j = t_j + b_all
        table_ref[pl.ds(SLOT_STRIDE * j, NUM_CARDS), :] = t_j


def _sc_body(table_hbm, hand_hbm, pub_hbm, out_hbm,
             table_v, hidx_v, pidx_v, out_v, sem):
    wid = lax.axis_index("s") * 2 + lax.axis_index("c")
    nh = 2 * BPW
    np_ = 5 * BPW

    c1 = pltpu.async_copy(table_hbm, table_v, sem)
    c2 = pltpu.async_copy(hand_hbm.at[pl.ds(wid * nh, nh)], hidx_v, sem)
    c3 = pltpu.async_copy(pub_hbm.at[pl.ds(wid * np_, np_)], pidx_v, sem)
    c1.wait()
    c2.wait()
    c3.wait()

    iota16 = lax.iota(jnp.int32, 16)
    ccs = [jnp.full((16,), c, jnp.int32) for c in range(NOUT)]

    @plsc.parallel_loop(0, BPW // 16, 1, unroll=2)
    def group(g):
        rows16 = iota16 + g * 16
        slots = [plsc.load_gather(hidx_v, [rows16 * 2]),
                 plsc.load_gather(hidx_v, [rows16 * 2 + 1]) + SLOT_STRIDE]
        for k in range(5):
            slots.append(plsc.load_gather(pidx_v, [rows16 * 5 + k])
                         + SLOT_STRIDE * (2 + k))
        for c in range(NOUT):
            acc = plsc.load_gather(table_v, [slots[0], ccs[c]])
            for s in slots[1:]:
                acc = acc + plsc.load_gather(table_v, [s, ccs[c]])
            plsc.store_scatter(out_v, [rows16, ccs[c]], acc)
    pltpu.sync_copy(out_v, out_hbm.at[pl.ds(wid * BPW, BPW)])


@functools.cache
def _sc_gather_sum():
    return pl.kernel(
        _sc_body,
        out_type=jax.ShapeDtypeStruct((B, OUT), jnp.float32),
        mesh=plsc.VectorSubcoreMesh(core_axis_name="c", subcore_axis_name="s"),
        scratch_types=[
            pltpu.VMEM((SLOT_STRIDE * 8, OUT), jnp.float32),
            pltpu.VMEM((2 * BPW,), jnp.int32),
            pltpu.VMEM((5 * BPW,), jnp.int32),
            pltpu.VMEM((BPW, OUT), jnp.float32),
            pltpu.SemaphoreType.DMA,
        ],
        compiler_params=pltpu.CompilerParams(use_tc_tiling_on_sc=False,
                                             needs_layout_passes=False,
                                             skip_device_barrier=True),
    )


def kernel(hand_indices, public_indices, embedding,
           W_ht1, b_ht1, W_ht2, b_ht2,
           W_wr1, b_wr1, W_wr2, b_wr2,
           W_ph1, b_ph1, W_ph2, b_ph2):
    i32 = jnp.int32
    hand_flat = hand_indices.astype(i32).reshape(-1)
    pub_flat = public_indices.astype(i32).reshape(-1)

    table = pl.pallas_call(
        _fold_body,
        out_shape=jax.ShapeDtypeStruct((SLOT_STRIDE * 8, OUT), jnp.float32),
    )(embedding, W_ht1, W_ht2, W_wr1, W_wr2, W_ph1, W_ph2,
      b_ht1.reshape(1, -1), b_wr1.reshape(1, -1), b_ph1.reshape(1, -1),
      jnp.concatenate([b_ht2, b_wr2, b_ph2,
                       jnp.zeros((4,), jnp.float32)]).reshape(1, OUT))

    out = _sc_gather_sum()(table, hand_flat, pub_flat)
    return (out[:, :10], out[:, 10:11], out[:, 11:12])


# speedup vs baseline: 1.1849x; 1.1530x over previous
import functools

import jax
import jax.numpy as jnp
from jax import lax
from jax.experimental import pallas as pl
from jax.experimental.pallas import tpu as pltpu
from jax.experimental.pallas import tpu_sc as plsc

B = 16384
NUM_CARDS = 52
EMB = 16
NOUT = 12
SLOT_STRIDE = 56
NSLOTS = 7
TBL = SLOT_STRIDE * 8
NW = 32
BPW = B // NW


def _fold_body(emb_ref, wht1_ref, wht2_ref, wwr1_ref, wwr2_ref,
               wph1_ref, wph2_ref, bht1_ref, bwr1_ref, bph1_ref, b2_ref,
               table_ref):
    f32 = jnp.float32
    i32 = jnp.int32

    r16_10 = lax.broadcasted_iota(i32, (16, 10), 0)
    c16_10 = lax.broadcasted_iota(i32, (16, 10), 1)
    s_ht = (r16_10 == c16_10).astype(f32)
    riota = lax.broadcasted_iota(i32, (16, 1024), 0)

    def matmul(a, b):
        return lax.dot_general(a, b, (((1,), (0,)), ((), ())),
                               precision=lax.Precision.HIGHEST,
                               preferred_element_type=f32)

    def matmul_t(a, b):
        return lax.dot_general(a, b, (((1,), (1,)), ((), ())),
                               precision=lax.Precision.HIGHEST,
                               preferred_element_type=f32)

    w2p_ht = matmul(s_ht, wht2_ref[...])
    w2p_wr = jnp.where(riota == 10,
                       jnp.broadcast_to(wwr2_ref[...], (16, 1024)), 0.0)
    w2p_ph = jnp.where(riota == 11,
                       jnp.broadcast_to(wph2_ref[...], (16, 1024)), 0.0)

    w_all = (matmul(w2p_ht, wht1_ref[...])
             + matmul(w2p_wr, wwr1_ref[...])
             + matmul(w2p_ph, wph1_ref[...]))

    b_all = (b2_ref[...]
             + matmul(w2p_ht, bht1_ref[...])
             + matmul(w2p_wr, bwr1_ref[...])
             + matmul(w2p_ph, bph1_ref[...]))

    emb = emb_ref[...]
    for j in range(NSLOTS):
        t_j = matmul_t(w_all[:, EMB * j:EMB * (j + 1)], emb)
        if j == 0:
            t_j = t_j + b_all
        table_ref[:, pl.ds(SLOT_STRIDE * j, NUM_CARDS)] = t_j


def _sc_body(table_hbm, hand_hbm, pub_hbm,
             hand_out, win_out, pot_out,
             table_v, hidx_v, pidx_v, hand_v, win_v, pot_v, sem):
    wid = lax.axis_index("s") * 2 + lax.axis_index("c")
    nh = 2 * BPW
    np_ = 5 * BPW

    c1 = pltpu.async_copy(table_hbm, table_v, sem)
    c2 = pltpu.async_copy(hand_hbm.at[pl.ds(wid * nh, nh)], hidx_v, sem)
    c3 = pltpu.async_copy(pub_hbm.at[pl.ds(wid * np_, np_)], pidx_v, sem)
    c1.wait()
    c2.wait()
    c3.wait()

    iota16 = lax.iota(jnp.int32, 16)

    @plsc.parallel_loop(0, BPW // 16, 1, unroll=2)
    def group(g):
        rows16 = iota16 + g * 16
        slots = [plsc.load_gather(hidx_v, [rows16 * 2]),
                 plsc.load_gather(hidx_v, [rows16 * 2 + 1]) + SLOT_STRIDE]
        for k in range(5):
            slots.append(plsc.load_gather(pidx_v, [rows16 * 5 + k])
                         + SLOT_STRIDE * (2 + k))
        rows10 = rows16 * 10
        for c in range(NOUT):
            acc = plsc.load_gather(table_v, [slots[0] + TBL * c])
            for s in slots[1:]:
                acc = acc + plsc.load_gather(table_v, [s + TBL * c])
            if c < 10:
                plsc.store_scatter(hand_v, [rows10 + c], acc)
            elif c == 10:
                plsc.store_scatter(win_v, [rows16], acc)
            else:
                plsc.store_scatter(pot_v, [rows16], acc)

    pltpu.sync_copy(hand_v, hand_out.at[pl.ds(wid * BPW * 10, BPW * 10)])
    pltpu.sync_copy(win_v, win_out.at[pl.ds(wid * BPW, BPW)])
    pltpu.sync_copy(pot_v, pot_out.at[pl.ds(wid * BPW, BPW)])


@functools.cache
def _sc_gather_sum():
    return pl.kernel(
        _sc_body,
        out_type=[jax.ShapeDtypeStruct((B * 10,), jnp.float32),
                  jax.ShapeDtypeStruct((B,), jnp.float32),
                  jax.ShapeDtypeStruct((B,), jnp.float32)],
        mesh=plsc.VectorSubcoreMesh(core_axis_name="c", subcore_axis_name="s"),
        scratch_types=[
            pltpu.VMEM((16 * TBL,), jnp.float32),
            pltpu.VMEM((2 * BPW,), jnp.int32),
            pltpu.VMEM((5 * BPW,), jnp.int32),
            pltpu.VMEM((BPW * 10,), jnp.float32),
            pltpu.VMEM((BPW,), jnp.float32),
            pltpu.VMEM((BPW,), jnp.float32),
            pltpu.SemaphoreType.DMA,
        ],
        compiler_params=pltpu.CompilerParams(use_tc_tiling_on_sc=False,
                                             needs_layout_passes=False,
                                             skip_device_barrier=True),
    )


def kernel(hand_indices, public_indices, embedding,
           W_ht1, b_ht1, W_ht2, b_ht2,
           W_wr1, b_wr1, W_wr2, b_wr2,
           W_ph1, b_ph1, W_ph2, b_ph2):
    i32 = jnp.int32
    hand_flat = hand_indices.astype(i32).reshape(-1)
    pub_flat = public_indices.astype(i32).reshape(-1)

    table = pl.pallas_call(
        _fold_body,
        out_shape=jax.ShapeDtypeStruct((16, TBL), jnp.float32),
    )(embedding, W_ht1, W_ht2, W_wr1, W_wr2, W_ph1, W_ph2,
      b_ht1.reshape(-1, 1), b_wr1.reshape(-1, 1), b_ph1.reshape(-1, 1),
      jnp.concatenate([b_ht2, b_wr2, b_ph2,
                       jnp.zeros((4,), jnp.float32)]).reshape(16, 1))

    hand, win, pot = _sc_gather_sum()(table.reshape(-1), hand_flat, pub_flat)
    return (hand.reshape(B, 10), win.reshape(B, 1), pot.reshape(B, 1))
